# baseline (speedup 1.0000x reference)
import jax
import jax.numpy as jnp
import numpy as np
from jax import lax
from jax.experimental import pallas as pl
from jax.experimental.pallas import tpu as pltpu
from jax.experimental.pallas import tpu_sc as plsc

N_FTS = 41024
D = 256
B = 16384
A = 30
NC = 2
NS = 16
NW = NC * NS
NB = 2 * B
BPW = NB // NW
G = 4
NG = BPW // G
R = G * A


def _embed_body(ics_hbm, table_hbm, out_hbm,
                idx_v, rows0, rows1, out0, out1,
                gsem0, gsem1, osem0, osem1):
    wid = lax.axis_index("s") * NC + lax.axis_index("c")

    pltpu.sync_copy(ics_hbm.at[wid], idx_v)

    def fire_gather(t, rows, gsem):
        pltpu.async_copy(table_hbm.at[idx_v.at[t]], rows, gsem)

    fire_gather(0, rows0, gsem0)
    fire_gather(1, rows1, gsem1)

    slots = ((rows0, out0, gsem0, osem0), (rows1, out1, gsem1, osem1))

    @pl.loop(0, NG, step=2)
    def _groups(t):
        for k, (rows, outb, gsem, osem) in enumerate(slots):
            tk = t + k
            pltpu.make_async_copy(table_hbm.at[idx_v.at[tk]], rows, gsem).wait()

            @pl.when(tk >= 2)
            def _():
                pltpu.make_async_copy(outb, out_hbm.at[wid * NG + tk], osem).wait()

            @pl.loop(0, G)
            def _elems(e):
                r0 = e * A

                def halves(a, c):
                    w = rows[r0 + a, pl.ds(c * 16, 16)]
                    lo = plsc.bitcast(jax.lax.shift_left(w, 16), jnp.float32)
                    hi = plsc.bitcast(
                        jax.lax.bitwise_and(w, jnp.int32(-65536)), jnp.float32)
                    return lo, hi

                for c in range(8):
                    l0, h0 = halves(0, c)
                    l1, h1 = halves(1, c)
                    for a in range(2, A, 2):
                        la, ha = halves(a, c)
                        l0 = l0 + la
                        h0 = h0 + ha
                        lb, hb = halves(a + 1, c)
                        l1 = l1 + lb
                        h1 = h1 + hb
                    outb[e, pl.ds(c * 32, 16)] = l0 + l1
                    outb[e, pl.ds(c * 32 + 16, 16)] = h0 + h1

            pltpu.async_copy(outb, out_hbm.at[wid * NG + tk], osem)

            @pl.when(tk + 2 < NG)
            def _():
                fire_gather(tk + 2, rows, gsem)

    pltpu.make_async_copy(out0, out_hbm.at[wid * NG + NG - 2], osem0).wait()
    pltpu.make_async_copy(out1, out_hbm.at[wid * NG + NG - 1], osem1).wait()


@jax.jit
def _embed_bag(ics, table):
    mesh = plsc.VectorSubcoreMesh(core_axis_name="c", subcore_axis_name="s")
    out = pl.kernel(
        _embed_body,
        out_type=jax.ShapeDtypeStruct((NW * NG, G, D), jnp.float32),
        mesh=mesh,
        scratch_types=[
            pltpu.VMEM((NG, R), jnp.int32),
            pltpu.VMEM((R, D // 2), jnp.int32),
            pltpu.VMEM((R, D // 2), jnp.int32),
            pltpu.VMEM((G, D), jnp.float32),
            pltpu.VMEM((G, D), jnp.float32),
            pltpu.SemaphoreType.DMA,
            pltpu.SemaphoreType.DMA,
            pltpu.SemaphoreType.DMA,
            pltpu.SemaphoreType.DMA,
        ],
        compiler_params=pltpu.CompilerParams(needs_layout_passes=False),
    )(ics, table)
    return out.reshape(NB, D)


def _psqt_body(icst_hbm, psq_hbm, out_hbm, icst_v, psq_v, acc_v):
    wid = lax.axis_index("s") * NC + lax.axis_index("c")
    base = pl.multiple_of(wid * (A * BPW), 8)
    pltpu.sync_copy(icst_hbm.at[pl.ds(base, A * BPW)], icst_v)
    pltpu.sync_copy(psq_hbm, psq_v)

    @pl.loop(0, BPW // 16)
    def _vecs(vg):
        e0 = pl.multiple_of(vg * 16, 16)
        iv = icst_v[pl.ds(e0, 16)]
        acc = plsc.load_gather(psq_v, [iv])
        for a in range(1, A):
            iv = icst_v[pl.ds(a * BPW + e0, 16)]
            acc = acc + plsc.load_gather(psq_v, [iv])
        acc_v[pl.ds(e0, 16)] = acc

    obase = pl.multiple_of(wid * BPW, 8)
    pltpu.sync_copy(acc_v, out_hbm.at[pl.ds(obase, BPW)])


@jax.jit
def _psqt(icst, psq):
    mesh = plsc.VectorSubcoreMesh(core_axis_name="c", subcore_axis_name="s")
    return pl.kernel(
        _psqt_body,
        out_type=jax.ShapeDtypeStruct((NB,), jnp.float32),
        mesh=mesh,
        scratch_types=[
            pltpu.VMEM((A * BPW,), jnp.int32),
            pltpu.VMEM((N_FTS,), jnp.float32),
            pltpu.VMEM((BPW,), jnp.float32),
        ],
        compiler_params=pltpu.CompilerParams(
            use_tc_tiling_on_sc=False, needs_layout_passes=False),
    )(icst, psq)


def _mlp_body(wref, bref, wpref, bpref, sref, ftb, w1, b1, w2, b2, wo, bo, oref):
    w256 = wref[...]
    b256 = bref[...]
    stm = sref[...]
    bias = ftb[...]
    first = jnp.clip((1.0 - stm) * w256 + stm * b256 + bias, 0.0, 1.0)
    second = jnp.clip((1.0 - stm) * b256 + stm * w256 + bias, 0.0, 1.0)
    fc1 = w1[...]
    dn = (((1,), (1,)), ((), ()))
    h = lax.dot_general(first, fc1[:, :256], dn, preferred_element_type=jnp.float32)
    h = h + lax.dot_general(second, fc1[:, 256:], dn, preferred_element_type=jnp.float32)
    h = jnp.clip(h + b1[...], 0.0, 1.0)
    h = jnp.clip(
        lax.dot_general(h, w2[...], dn, preferred_element_type=jnp.float32) + b2[...],
        0.0, 1.0)
    o = jnp.sum(h * wo[...], axis=1, keepdims=True) + bo[0, 0]
    o = o + (wpref[...] - bpref[...]) * (0.5 - stm)
    oref[...] = o


_MLP_BLK = 512


@jax.jit
def _mlp(fts, psq, stm, ftb, fc1_w, fc1_b, fc2_w, fc2_b, fco_w, fco_b):
    nblk = B // _MLP_BLK
    full = lambda shape: pl.BlockSpec(shape, lambda i: (0, 0))
    return pl.pallas_call(
        _mlp_body,
        grid=(nblk,),
        in_specs=[
            pl.BlockSpec((_MLP_BLK, D), lambda i: (i, 0)),
            pl.BlockSpec((_MLP_BLK, D), lambda i: (i + nblk, 0)),
            pl.BlockSpec((_MLP_BLK, 1), lambda i: (i, 0)),
            pl.BlockSpec((_MLP_BLK, 1), lambda i: (i + nblk, 0)),
            pl.BlockSpec((_MLP_BLK, 1), lambda i: (i, 0)),
            full((1, D)),
            full((32, 512)),
            full((1, 32)),
            full((32, 32)),
            full((1, 32)),
            full((1, 32)),
            pl.BlockSpec(memory_space=pltpu.SMEM),
        ],
        out_specs=pl.BlockSpec((_MLP_BLK, 1), lambda i: (i, 0)),
        out_shape=jax.ShapeDtypeStruct((B, 1), jnp.float32),
        compiler_params=pltpu.CompilerParams(
            dimension_semantics=("arbitrary",),
        ),
    )(fts, fts, psq, psq, stm, ftb, fc1_w, fc1_b, fc2_w, fc2_b, fco_w, fco_b)


_PERM = np.arange(D).reshape(8, 2, 16)
_PERM = (_PERM[:, :1, :] * 0 + np.arange(16) * 2 +
         np.arange(2)[None, :, None] + (np.arange(8) * 32)[:, None, None])
_PERM = _PERM.reshape(D).astype(np.int32)


def kernel(wft_ics, bft_ics, stm, ft_w, ft_b, fc1_w, fc1_b, fc2_w, fc2_b,
           fco_w, fco_b):
    ics = jnp.concatenate([wft_ics, bft_ics], axis=0)
    fw_i = jax.lax.bitcast_convert_type(ft_w, jnp.int32)
    ev = fw_i[:, 0:D:2] + jnp.int32(0x8000)
    od = fw_i[:, 1:D:2] + jnp.int32(0x8000)
    tableA = jax.lax.shift_right_logical(ev, 16) | (od & jnp.int32(-65536))
    psqcol = ft_w[:, D]
    fts = _embed_bag(ics.reshape(NW, NG, R), tableA)
    icst = ics.reshape(NW, BPW, A).transpose(0, 2, 1).reshape(-1)
    psq = _psqt(icst, psqcol)
    fc1p = jnp.concatenate([fc1_w[:, :D][:, _PERM], fc1_w[:, D:][:, _PERM]],
                           axis=1)
    ftbp = ft_b[:D][_PERM].reshape(1, D)
    return _mlp(fts, psq.reshape(NB, 1), stm, ftbp, fc1p,
                fc1_b.reshape(1, 32), fc2_w, fc2_b.reshape(1, 32), fco_w,
                fco_b.reshape(1, 1))


# speedup vs baseline: 3.0431x; 3.0431x over previous
import jax
import jax.numpy as jnp
import numpy as np
from jax import lax
from jax.experimental import pallas as pl
from jax.experimental.pallas import tpu as pltpu
from jax.experimental.pallas import tpu_sc as plsc

N_FTS = 41024
D = 256
B = 16384
A = 30
NC = 2
NS = 16
NW = NC * NS
NB = 2 * B
BPW = NB // NW
G = 4
NG = BPW // G
R = G * A


def _embed_body(ics_hbm, table_hbm, out_hbm,
                idx_v, rows0, rows1, out0, out1,
                gsem0, gsem1, osem0, osem1):
    wid = lax.axis_index("s") * NC + lax.axis_index("c")

    pltpu.sync_copy(ics_hbm.at[wid], idx_v)

    def fire_gather(t, rows, gsem):
        pltpu.async_copy(table_hbm.at[idx_v.at[t]], rows, gsem)

    fire_gather(0, rows0, gsem0)
    fire_gather(1, rows1, gsem1)

    slots = ((rows0, out0, gsem0, osem0), (rows1, out1, gsem1, osem1))

    @pl.loop(0, NG, step=2)
    def _groups(t):
        for k, (rows, outb, gsem, osem) in enumerate(slots):
            tk = t + k
            pltpu.make_async_copy(table_hbm.at[idx_v.at[tk]], rows, gsem).wait()

            @pl.when(tk >= 2)
            def _():
                pltpu.make_async_copy(outb, out_hbm.at[wid * NG + tk], osem).wait()

            @pl.loop(0, G)
            def _elems(e):
                r0 = e * A

                def halves(a, c):
                    w = rows[r0 + a, pl.ds(c * 16, 16)]
                    lo = plsc.bitcast(jax.lax.shift_left(w, 16), jnp.float32)
                    hi = plsc.bitcast(
                        jax.lax.bitwise_and(w, jnp.int32(-65536)), jnp.float32)
                    return lo, hi

                for c in range(8):
                    l0, h0 = halves(0, c)
                    l1, h1 = halves(1, c)
                    for a in range(2, A, 2):
                        la, ha = halves(a, c)
                        l0 = l0 + la
                        h0 = h0 + ha
                        lb, hb = halves(a + 1, c)
                        l1 = l1 + lb
                        h1 = h1 + hb
                    outb[e, pl.ds(c * 16, 16)] = l0 + l1
                    outb[e, pl.ds(128 + c * 16, 16)] = h0 + h1

            pltpu.async_copy(outb, out_hbm.at[wid * NG + tk], osem)

            @pl.when(tk + 2 < NG)
            def _():
                fire_gather(tk + 2, rows, gsem)

    pltpu.make_async_copy(out0, out_hbm.at[wid * NG + NG - 2], osem0).wait()
    pltpu.make_async_copy(out1, out_hbm.at[wid * NG + NG - 1], osem1).wait()


@jax.jit
def _embed_bag(ics, table):
    mesh = plsc.VectorSubcoreMesh(core_axis_name="c", subcore_axis_name="s")
    out = pl.kernel(
        _embed_body,
        out_type=jax.ShapeDtypeStruct((NW * NG, G, D), jnp.float32),
        mesh=mesh,
        scratch_types=[
            pltpu.VMEM((NG, R), jnp.int32),
            pltpu.VMEM((R, D // 2), jnp.int32),
            pltpu.VMEM((R, D // 2), jnp.int32),
            pltpu.VMEM((G, D), jnp.float32),
            pltpu.VMEM((G, D), jnp.float32),
            pltpu.SemaphoreType.DMA,
            pltpu.SemaphoreType.DMA,
            pltpu.SemaphoreType.DMA,
            pltpu.SemaphoreType.DMA,
        ],
        compiler_params=pltpu.CompilerParams(needs_layout_passes=False),
    )(ics, table)
    return out.reshape(NB, D)


def _psqt_body(icst_hbm, psq_hbm, out_hbm, icst_v, psq_v, acc_v):
    wid = lax.axis_index("s") * NC + lax.axis_index("c")
    base = pl.multiple_of(wid * (A * BPW), 8)
    pltpu.sync_copy(icst_hbm.at[pl.ds(base, A * BPW)], icst_v)
    pltpu.sync_copy(psq_hbm, psq_v)

    @pl.loop(0, BPW // 16)
    def _vecs(vg):
        e0 = pl.multiple_of(vg * 16, 16)
        iv = icst_v[pl.ds(e0, 16)]
        acc = plsc.load_gather(psq_v, [iv])
        for a in range(1, A):
            iv = icst_v[pl.ds(a * BPW + e0, 16)]
            acc = acc + plsc.load_gather(psq_v, [iv])
        acc_v[pl.ds(e0, 16)] = acc

    obase = pl.multiple_of(wid * BPW, 8)
    pltpu.sync_copy(acc_v, out_hbm.at[pl.ds(obase, BPW)])


@jax.jit
def _psqt(icst, psq):
    mesh = plsc.VectorSubcoreMesh(core_axis_name="c", subcore_axis_name="s")
    return pl.kernel(
        _psqt_body,
        out_type=jax.ShapeDtypeStruct((NB,), jnp.float32),
        mesh=mesh,
        scratch_types=[
            pltpu.VMEM((A * BPW,), jnp.int32),
            pltpu.VMEM((N_FTS,), jnp.float32),
            pltpu.VMEM((BPW,), jnp.float32),
        ],
        compiler_params=pltpu.CompilerParams(
            use_tc_tiling_on_sc=False, needs_layout_passes=False),
    )(icst, psq)


def _mlp_body(wref, bref, wpref, bpref, sref, ftb, w1, b1, w2, b2, wo, bo, oref):
    w256 = wref[...]
    b256 = bref[...]
    stm = sref[...]
    bias = ftb[...]
    first = jnp.clip((1.0 - stm) * w256 + stm * b256 + bias, 0.0, 1.0)
    second = jnp.clip((1.0 - stm) * b256 + stm * w256 + bias, 0.0, 1.0)
    fc1 = w1[...]
    dn = (((1,), (1,)), ((), ()))
    h = lax.dot_general(first, fc1[:, :256], dn, preferred_element_type=jnp.float32)
    h = h + lax.dot_general(second, fc1[:, 256:], dn, preferred_element_type=jnp.float32)
    h = jnp.clip(h + b1[...], 0.0, 1.0)
    h = jnp.clip(
        lax.dot_general(h, w2[...], dn, preferred_element_type=jnp.float32) + b2[...],
        0.0, 1.0)
    o = jnp.sum(h * wo[...], axis=1, keepdims=True) + bo[0, 0]
    o = o + (wpref[...] - bpref[...]) * (0.5 - stm)
    oref[...] = o


_MLP_BLK = 512


@jax.jit
def _mlp(fts, psq, stm, ftb, fc1_w, fc1_b, fc2_w, fc2_b, fco_w, fco_b):
    nblk = B // _MLP_BLK
    full = lambda shape: pl.BlockSpec(shape, lambda i: (0, 0))
    return pl.pallas_call(
        _mlp_body,
        grid=(nblk,),
        in_specs=[
            pl.BlockSpec((_MLP_BLK, D), lambda i: (i, 0)),
            pl.BlockSpec((_MLP_BLK, D), lambda i: (i + nblk, 0)),
            pl.BlockSpec((_MLP_BLK, 1), lambda i: (i, 0)),
            pl.BlockSpec((_MLP_BLK, 1), lambda i: (i + nblk, 0)),
            pl.BlockSpec((_MLP_BLK, 1), lambda i: (i, 0)),
            full((1, D)),
            full((32, 512)),
            full((1, 32)),
            full((32, 32)),
            full((1, 32)),
            full((1, 32)),
            pl.BlockSpec(memory_space=pltpu.SMEM),
        ],
        out_specs=pl.BlockSpec((_MLP_BLK, 1), lambda i: (i, 0)),
        out_shape=jax.ShapeDtypeStruct((B, 1), jnp.float32),
        compiler_params=pltpu.CompilerParams(
            dimension_semantics=("arbitrary",),
        ),
    )(fts, fts, psq, psq, stm, ftb, fc1_w, fc1_b, fc2_w, fc2_b, fco_w, fco_b)


def kernel(wft_ics, bft_ics, stm, ft_w, ft_b, fc1_w, fc1_b, fc2_w, fc2_b,
           fco_w, fco_b):
    ics = jnp.concatenate([wft_ics, bft_ics], axis=0)
    fw_i = jax.lax.bitcast_convert_type(ft_w, jnp.int32)
    ev = fw_i[:, : D // 2] + jnp.int32(0x8000)
    od = fw_i[:, D // 2 : D] + jnp.int32(0x8000)
    tableA = jax.lax.shift_right_logical(ev, 16) | (od & jnp.int32(-65536))
    psqcol = ft_w[:, D]
    fts = _embed_bag(ics.reshape(NW, NG, R), tableA)
    icst = ics.reshape(NW, BPW, A).transpose(0, 2, 1).reshape(-1)
    psq = _psqt(icst, psqcol)
    return _mlp(fts, psq.reshape(NB, 1), stm, ft_b[:D].reshape(1, D), fc1_w,
                fc1_b.reshape(1, 32), fc2_w, fc2_b.reshape(1, 32), fco_w,
                fco_b.reshape(1, 1))


# speedup vs baseline: 3.0991x; 1.0184x over previous
import jax
import jax.numpy as jnp
import numpy as np
from jax import lax
from jax.experimental import pallas as pl
from jax.experimental.pallas import tpu as pltpu
from jax.experimental.pallas import tpu_sc as plsc

N_FTS = 41024
D = 256
B = 16384
A = 30
NC = 2
NS = 16
NW = NC * NS
NB = 2 * B
BPW = NB // NW
G = 4
NG = BPW // G
R = G * A


def _embed_body(ics_hbm, table_hbm, out_hbm,
                idx_v, rows0, rows1, out0, out1,
                gsem0, gsem1, osem0, osem1):
    wid = lax.axis_index("s") * NC + lax.axis_index("c")

    pltpu.sync_copy(ics_hbm.at[wid], idx_v)

    def fire_gather(t, rows, gsem):
        pltpu.async_copy(table_hbm.at[idx_v.at[t]], rows, gsem)

    fire_gather(0, rows0, gsem0)
    fire_gather(1, rows1, gsem1)

    slots = ((rows0, out0, gsem0, osem0), (rows1, out1, gsem1, osem1))

    @pl.loop(0, NG, step=2)
    def _groups(t):
        for k, (rows, outb, gsem, osem) in enumerate(slots):
            tk = t + k
            pltpu.make_async_copy(table_hbm.at[idx_v.at[tk]], rows, gsem).wait()

            @pl.when(tk >= 2)
            def _():
                pltpu.make_async_copy(outb, out_hbm.at[wid * NG + tk], osem).wait()

            @pl.loop(0, G)
            def _elems(e):
                r0 = e * A

                def halves(a, c):
                    w = rows[r0 + a, pl.ds(c * 16, 16)]
                    lo = plsc.bitcast(jax.lax.shift_left(w, 16), jnp.float32)
                    hi = plsc.bitcast(
                        jax.lax.bitwise_and(w, jnp.int32(-65536)), jnp.float32)
                    return lo, hi

                for c in range(8):
                    l0, h0 = halves(0, c)
                    l1, h1 = halves(1, c)
                    for a in range(2, A, 2):
                        la, ha = halves(a, c)
                        l0 = l0 + la
                        h0 = h0 + ha
                        lb, hb = halves(a + 1, c)
                        l1 = l1 + lb
                        h1 = h1 + hb
                    outb[e, pl.ds(c * 16, 16)] = l0 + l1
                    outb[e, pl.ds(128 + c * 16, 16)] = h0 + h1

            pltpu.async_copy(outb, out_hbm.at[wid * NG + tk], osem)

            @pl.when(tk + 2 < NG)
            def _():
                fire_gather(tk + 2, rows, gsem)

    pltpu.make_async_copy(out0, out_hbm.at[wid * NG + NG - 2], osem0).wait()
    pltpu.make_async_copy(out1, out_hbm.at[wid * NG + NG - 1], osem1).wait()


@jax.jit
def _embed_bag(ics, table):
    mesh = plsc.VectorSubcoreMesh(core_axis_name="c", subcore_axis_name="s")
    out = pl.kernel(
        _embed_body,
        out_type=jax.ShapeDtypeStruct((NW * NG, G, D), jnp.float32),
        mesh=mesh,
        scratch_types=[
            pltpu.VMEM((NG, R), jnp.int32),
            pltpu.VMEM((R, D // 2), jnp.int32),
            pltpu.VMEM((R, D // 2), jnp.int32),
            pltpu.VMEM((G, D), jnp.float32),
            pltpu.VMEM((G, D), jnp.float32),
            pltpu.SemaphoreType.DMA,
            pltpu.SemaphoreType.DMA,
            pltpu.SemaphoreType.DMA,
            pltpu.SemaphoreType.DMA,
        ],
        compiler_params=pltpu.CompilerParams(needs_layout_passes=False),
    )(ics, table)
    return out.reshape(NB, D)


def _psqt_body(icst_hbm, psq_hbm, out_hbm, icst_v, psq_v, acc_v):
    wid = lax.axis_index("s") * NC + lax.axis_index("c")
    base = pl.multiple_of(wid * (A * BPW), 8)
    pltpu.sync_copy(icst_hbm.at[pl.ds(base, A * BPW)], icst_v)
    pltpu.sync_copy(psq_hbm, psq_v)

    @pl.loop(0, BPW // 16)
    def _vecs(vg):
        e0 = pl.multiple_of(vg * 16, 16)
        iv = icst_v[pl.ds(e0, 16)]
        acc = plsc.load_gather(psq_v, [iv])
        for a in range(1, A):
            iv = icst_v[pl.ds(a * BPW + e0, 16)]
            acc = acc + plsc.load_gather(psq_v, [iv])
        acc_v[pl.ds(e0, 16)] = acc

    obase = pl.multiple_of(wid * BPW, 8)
    pltpu.sync_copy(acc_v, out_hbm.at[pl.ds(obase, BPW)])


@jax.jit
def _psqt(icst, psq):
    mesh = plsc.VectorSubcoreMesh(core_axis_name="c", subcore_axis_name="s")
    return pl.kernel(
        _psqt_body,
        out_type=jax.ShapeDtypeStruct((NB,), jnp.float32),
        mesh=mesh,
        scratch_types=[
            pltpu.VMEM((A * BPW,), jnp.int32),
            pltpu.VMEM((N_FTS,), jnp.float32),
            pltpu.VMEM((BPW,), jnp.float32),
        ],
        compiler_params=pltpu.CompilerParams(
            use_tc_tiling_on_sc=False, needs_layout_passes=False),
    )(icst, psq)


def _mlp_body(wref, bref, wpref, bpref, sref, ftb, w1, b1, w2, b2, wo, bo, oref):
    w256 = wref[...]
    b256 = bref[...]
    stm = sref[...]
    bias = ftb[...]
    first = jnp.clip((1.0 - stm) * w256 + stm * b256 + bias, 0.0, 1.0)
    second = jnp.clip((1.0 - stm) * b256 + stm * w256 + bias, 0.0, 1.0)
    fc1 = w1[...]
    dn = (((1,), (1,)), ((), ()))
    h = lax.dot_general(first, fc1[:, :256], dn, preferred_element_type=jnp.float32)
    h = h + lax.dot_general(second, fc1[:, 256:], dn, preferred_element_type=jnp.float32)
    h = jnp.clip(h + b1[...], 0.0, 1.0)
    h = jnp.clip(
        lax.dot_general(h, w2[...], dn, preferred_element_type=jnp.float32) + b2[...],
        0.0, 1.0)
    o = jnp.sum(h * wo[...], axis=1, keepdims=True) + bo[0, 0]
    o = o + (wpref[...] - bpref[...]) * (0.5 - stm)
    oref[...] = o


_MLP_BLK = 512


@jax.jit
def _mlp(fts, psq, stm, ftb, fc1_w, fc1_b, fc2_w, fc2_b, fco_w, fco_b):
    nblk = B // _MLP_BLK
    full = lambda shape: pl.BlockSpec(shape, lambda i: (0, 0))
    return pl.pallas_call(
        _mlp_body,
        grid=(nblk,),
        in_specs=[
            pl.BlockSpec((_MLP_BLK, D), lambda i: (i, 0)),
            pl.BlockSpec((_MLP_BLK, D), lambda i: (i + nblk, 0)),
            pl.BlockSpec((_MLP_BLK, 1), lambda i: (i, 0)),
            pl.BlockSpec((_MLP_BLK, 1), lambda i: (i + nblk, 0)),
            pl.BlockSpec((_MLP_BLK, 1), lambda i: (i, 0)),
            full((1, D)),
            full((32, 512)),
            full((1, 32)),
            full((32, 32)),
            full((1, 32)),
            full((1, 32)),
            pl.BlockSpec(memory_space=pltpu.SMEM),
        ],
        out_specs=pl.BlockSpec((_MLP_BLK, 1), lambda i: (i, 0)),
        out_shape=jax.ShapeDtypeStruct((B, 1), jnp.float32),
        compiler_params=pltpu.CompilerParams(
            dimension_semantics=("arbitrary",),
        ),
    )(fts, fts, psq, psq, stm, ftb, fc1_w, fc1_b, fc2_w, fc2_b, fco_w, fco_b)


def kernel(wft_ics, bft_ics, stm, ft_w, ft_b, fc1_w, fc1_b, fc2_w, fc2_b,
           fco_w, fco_b):
    ics = jnp.concatenate([wft_ics, bft_ics], axis=0)
    fw_i = jax.lax.bitcast_convert_type(ft_w.T, jnp.int32)
    ev = fw_i[: D // 2] + jnp.int32(0x8000)
    od = fw_i[D // 2 : D] + jnp.int32(0x8000)
    tableA = (jax.lax.shift_right_logical(ev, 16) | (od & jnp.int32(-65536))).T
    psqcol = ft_w[:, D]
    fts = _embed_bag(ics.reshape(NW, NG, R), tableA)
    icst = ics.reshape(NW, BPW, A).transpose(0, 2, 1).reshape(-1)
    psq = _psqt(icst, psqcol)
    return _mlp(fts, psq.reshape(NB, 1), stm, ft_b[:D].reshape(1, D), fc1_w,
                fc1_b.reshape(1, 32), fc2_w, fc2_b.reshape(1, 32), fco_w,
                fco_b.reshape(1, 1))


# speedup vs baseline: 3.6263x; 1.1701x over previous
import jax
import jax.numpy as jnp
import numpy as np
from jax import lax
from jax.experimental import pallas as pl
from jax.experimental.pallas import tpu as pltpu
from jax.experimental.pallas import tpu_sc as plsc

N_FTS = 41024
D = 256
B = 16384
A = 30
NC = 2
NS = 16
NW = NC * NS
NB = 2 * B
BPW = NB // NW
G = 4
NG = BPW // G
R = G * A


def _embed_body(ics_hbm, table_hbm, out_hbm,
                idx_v, rows0, rows1, out0, out1,
                gsem0, gsem1, osem0, osem1):
    wid = lax.axis_index("s") * NC + lax.axis_index("c")

    pltpu.sync_copy(ics_hbm.at[wid], idx_v)

    def fire_gather(t, rows, gsem):
        pltpu.async_copy(table_hbm.at[idx_v.at[t]], rows, gsem)

    fire_gather(0, rows0, gsem0)
    fire_gather(1, rows1, gsem1)

    slots = ((rows0, out0, gsem0, osem0), (rows1, out1, gsem1, osem1))

    @pl.loop(0, NG, step=2)
    def _groups(t):
        for k, (rows, outb, gsem, osem) in enumerate(slots):
            tk = t + k
            pltpu.make_async_copy(table_hbm.at[idx_v.at[tk]], rows, gsem).wait()

            @pl.when(tk >= 2)
            def _():
                pltpu.make_async_copy(outb, out_hbm.at[wid * NG + tk], osem).wait()

            @pl.loop(0, G)
            def _elems(e):
                r0 = e * A

                def halves(a, c):
                    w = rows[r0 + a, pl.ds(c * 16, 16)]
                    lo = plsc.bitcast(jax.lax.shift_left(w, 16), jnp.float32)
                    hi = plsc.bitcast(
                        jax.lax.bitwise_and(w, jnp.int32(-65536)), jnp.float32)
                    return lo, hi

                for c in range(8):
                    l0, h0 = halves(0, c)
                    l1, h1 = halves(1, c)
                    for a in range(2, A, 2):
                        la, ha = halves(a, c)
                        l0 = l0 + la
                        h0 = h0 + ha
                        lb, hb = halves(a + 1, c)
                        l1 = l1 + lb
                        h1 = h1 + hb
                    outb[e, pl.ds(c * 16, 16)] = l0 + l1
                    outb[e, pl.ds(128 + c * 16, 16)] = h0 + h1

            pltpu.async_copy(outb, out_hbm.at[wid * NG + tk], osem)

            @pl.when(tk + 2 < NG)
            def _():
                fire_gather(tk + 2, rows, gsem)

    pltpu.make_async_copy(out0, out_hbm.at[wid * NG + NG - 2], osem0).wait()
    pltpu.make_async_copy(out1, out_hbm.at[wid * NG + NG - 1], osem1).wait()


@jax.jit
def _embed_bag(ics, table):
    mesh = plsc.VectorSubcoreMesh(core_axis_name="c", subcore_axis_name="s")
    out = pl.kernel(
        _embed_body,
        out_type=jax.ShapeDtypeStruct((NW * NG, G, D), jnp.float32),
        mesh=mesh,
        scratch_types=[
            pltpu.VMEM((NG, R), jnp.int32),
            pltpu.VMEM((R, D // 2), jnp.int32),
            pltpu.VMEM((R, D // 2), jnp.int32),
            pltpu.VMEM((G, D), jnp.float32),
            pltpu.VMEM((G, D), jnp.float32),
            pltpu.SemaphoreType.DMA,
            pltpu.SemaphoreType.DMA,
            pltpu.SemaphoreType.DMA,
            pltpu.SemaphoreType.DMA,
        ],
        compiler_params=pltpu.CompilerParams(needs_layout_passes=False),
    )(ics, table)
    return out.reshape(NB, D)


def _psqt_body(icst_hbm, psq_hbm, out_hbm, icst_v, psq_v, acc_v):
    wid = lax.axis_index("s") * NC + lax.axis_index("c")
    base = pl.multiple_of(wid * (A * BPW), 8)
    pltpu.sync_copy(icst_hbm.at[pl.ds(base, A * BPW)], icst_v)
    pltpu.sync_copy(psq_hbm, psq_v)

    @pl.loop(0, BPW // 16)
    def _vecs(vg):
        e0 = pl.multiple_of(vg * 16, 16)
        iv = icst_v[pl.ds(e0, 16)]
        acc = plsc.load_gather(psq_v, [iv])
        for a in range(1, A):
            iv = icst_v[pl.ds(a * BPW + e0, 16)]
            acc = acc + plsc.load_gather(psq_v, [iv])
        acc_v[pl.ds(e0, 16)] = acc

    obase = pl.multiple_of(wid * BPW, 8)
    pltpu.sync_copy(acc_v, out_hbm.at[pl.ds(obase, BPW)])


@jax.jit
def _psqt(icst, psq):
    mesh = plsc.VectorSubcoreMesh(core_axis_name="c", subcore_axis_name="s")
    return pl.kernel(
        _psqt_body,
        out_type=jax.ShapeDtypeStruct((NB,), jnp.float32),
        mesh=mesh,
        scratch_types=[
            pltpu.VMEM((A * BPW,), jnp.int32),
            pltpu.VMEM((N_FTS,), jnp.float32),
            pltpu.VMEM((BPW,), jnp.float32),
        ],
        compiler_params=pltpu.CompilerParams(
            use_tc_tiling_on_sc=False, needs_layout_passes=False),
    )(icst, psq)


def _tr_body(xref, oref):
    oref[...] = xref[...].T


@jax.jit
def _transpose_pack(packed_t):
    blk = 512
    return pl.pallas_call(
        _tr_body,
        grid=(N_FTS // blk,),
        in_specs=[pl.BlockSpec((D // 2, blk), lambda j: (0, j))],
        out_specs=pl.BlockSpec((blk, D // 2), lambda j: (j, 0)),
        out_shape=jax.ShapeDtypeStruct((N_FTS, D // 2), jnp.int32),
        compiler_params=pltpu.CompilerParams(
            dimension_semantics=("arbitrary",),
        ),
    )(packed_t)


def _mlp_body(wref, bref, wpref, bpref, sref, ftb, w1, b1, w2, b2, wo, bo, oref):
    w256 = wref[...]
    b256 = bref[...]
    stm = sref[...]
    bias = ftb[...]
    first = jnp.clip((1.0 - stm) * w256 + stm * b256 + bias, 0.0, 1.0)
    second = jnp.clip((1.0 - stm) * b256 + stm * w256 + bias, 0.0, 1.0)
    fc1 = w1[...]
    dn = (((1,), (1,)), ((), ()))
    h = lax.dot_general(first, fc1[:, :256], dn, preferred_element_type=jnp.float32)
    h = h + lax.dot_general(second, fc1[:, 256:], dn, preferred_element_type=jnp.float32)
    h = jnp.clip(h + b1[...], 0.0, 1.0)
    h = jnp.clip(
        lax.dot_general(h, w2[...], dn, preferred_element_type=jnp.float32) + b2[...],
        0.0, 1.0)
    o = jnp.sum(h * wo[...], axis=1, keepdims=True) + bo[0, 0]
    o = o + (wpref[...] - bpref[...]) * (0.5 - stm)
    oref[...] = o


_MLP_BLK = 512


@jax.jit
def _mlp(fts, psq, stm, ftb, fc1_w, fc1_b, fc2_w, fc2_b, fco_w, fco_b):
    nblk = B // _MLP_BLK
    full = lambda shape: pl.BlockSpec(shape, lambda i: (0, 0))
    return pl.pallas_call(
        _mlp_body,
        grid=(nblk,),
        in_specs=[
            pl.BlockSpec((_MLP_BLK, D), lambda i: (i, 0)),
            pl.BlockSpec((_MLP_BLK, D), lambda i: (i + nblk, 0)),
            pl.BlockSpec((_MLP_BLK, 1), lambda i: (i, 0)),
            pl.BlockSpec((_MLP_BLK, 1), lambda i: (i + nblk, 0)),
            pl.BlockSpec((_MLP_BLK, 1), lambda i: (i, 0)),
            full((1, D)),
            full((32, 512)),
            full((1, 32)),
            full((32, 32)),
            full((1, 32)),
            full((1, 32)),
            pl.BlockSpec(memory_space=pltpu.SMEM),
        ],
        out_specs=pl.BlockSpec((_MLP_BLK, 1), lambda i: (i, 0)),
        out_shape=jax.ShapeDtypeStruct((B, 1), jnp.float32),
        compiler_params=pltpu.CompilerParams(
            dimension_semantics=("arbitrary",),
        ),
    )(fts, fts, psq, psq, stm, ftb, fc1_w, fc1_b, fc2_w, fc2_b, fco_w, fco_b)


def kernel(wft_ics, bft_ics, stm, ft_w, ft_b, fc1_w, fc1_b, fc2_w, fc2_b,
           fco_w, fco_b):
    ics = jnp.concatenate([wft_ics, bft_ics], axis=0)
    fw_i = jax.lax.bitcast_convert_type(ft_w.T, jnp.int32)
    ev = fw_i[: D // 2] + jnp.int32(0x8000)
    od = fw_i[D // 2 : D] + jnp.int32(0x8000)
    packed_t = jax.lax.shift_right_logical(ev, 16) | (od & jnp.int32(-65536))
    tableA = _transpose_pack(packed_t)
    psqcol = ft_w[:, D]
    fts = _embed_bag(ics.reshape(NW, NG, R), tableA)
    icst = ics.reshape(NW, BPW, A).transpose(0, 2, 1).reshape(-1)
    psq = _psqt(icst, psqcol)
    return _mlp(fts, psq.reshape(NB, 1), stm, ft_b[:D].reshape(1, D), fc1_w,
                fc1_b.reshape(1, 32), fc2_w, fc2_b.reshape(1, 32), fco_w,
                fco_b.reshape(1, 1))
